# Initial kernel scaffold; baseline (speedup 1.0000x reference)
import jax
import jax.numpy as jnp
from jax.experimental import pallas as pl


def kernel(query, values, index, W):
    raise NotImplementedError("write your pallas kernel here")


# speedup vs baseline: 13.2680x; 13.2680x over previous
import functools
import math

import numpy as np
import jax
import jax.numpy as jnp
from jax import lax
from jax.experimental import pallas as pl
from jax.experimental.pallas import tpu as pltpu

DEC = 32
ENC = 128
N_NODES = 320000
SEGS = 64

NB = 2560
NBLK = N_NODES // NB
NEG = -1e30


def _attn_body(vals_ref, idx_ref, q_ref, w_ref, out_ref, m_scr, d_scr, a_scr):
    i = pl.program_id(0)

    @pl.when(i == 0)
    def _init():
        m_scr[...] = jnp.full((1, SEGS), NEG, jnp.float32)
        d_scr[...] = jnp.zeros((1, SEGS), jnp.float32)
        a_scr[...] = jnp.zeros((ENC, SEGS), jnp.float32)

    vals = vals_ref[...]
    idx = idx_ref[0, 0, :]
    onehot = idx[:, None] == lax.broadcasted_iota(jnp.int32, (NB, SEGS), 1)

    t = lax.dot_general(
        vals, w_ref[...],
        dimension_numbers=(((1,), (1,)), ((), ())),
        preferred_element_type=jnp.float32,
    )
    scores = lax.dot_general(
        t, q_ref[...],
        dimension_numbers=(((1,), (1,)), ((), ())),
        preferred_element_type=jnp.float32,
    ) / np.sqrt(DEC)
    smask = jnp.where(onehot, scores, NEG)
    bm = jnp.max(smask, axis=0, keepdims=True)
    m_old = m_scr[...]
    m_new = jnp.maximum(m_old, bm)
    scale = jnp.exp(m_old - m_new)
    p = jnp.where(onehot, jnp.exp(smask - m_new), 0.0)
    m_scr[...] = m_new
    d_scr[...] = d_scr[...] * scale + jnp.sum(p, axis=0, keepdims=True)
    contrib = lax.dot_general(
        vals, p,
        dimension_numbers=(((0,), (0,)), ((), ())),
        preferred_element_type=jnp.float32,
        precision=lax.Precision.HIGHEST,
    )
    a_scr[...] = a_scr[...] * scale + contrib

    @pl.when(i == NBLK - 1)
    def _emit():
        out_ref[...] = a_scr[...] / (d_scr[...] + 1e-16)


@functools.partial(jax.jit, static_argnames=("interpret",))
def kernel(query, values, index, W, interpret=False):
    idx3 = index.reshape(NBLK, 1, NB)
    out_t = pl.pallas_call(
        _attn_body,
        grid=(NBLK,),
        in_specs=[
            pl.BlockSpec((NB, ENC), lambda i: (i, 0)),
            pl.BlockSpec((1, 1, NB), lambda i: (i, 0, 0)),
            pl.BlockSpec((SEGS, DEC), lambda i: (0, 0)),
            pl.BlockSpec((DEC, ENC), lambda i: (0, 0)),
        ],
        out_specs=pl.BlockSpec((ENC, SEGS), lambda i: (0, 0)),
        out_shape=jax.ShapeDtypeStruct((ENC, SEGS), jnp.float32),
        scratch_shapes=[
            pltpu.VMEM((1, SEGS), jnp.float32),
            pltpu.VMEM((1, SEGS), jnp.float32),
            pltpu.VMEM((ENC, SEGS), jnp.float32),
        ],
        interpret=interpret,
    )(values, idx3, query, W)
    return out_t.T
